# baseline (speedup 1.0000x reference)
import functools

import jax
import jax.numpy as jnp
from jax import lax
from jax.experimental import pallas as pl
from jax.experimental.pallas import tpu as pltpu
from jax.experimental.pallas import tpu_sc as plsc

_VOCAB = 200
_HID = 1024
_PHID = 512
_OUT_W = 49152
_N_ROWS = 6400
_CB = 2048
_N_CB = _OUT_W // _CB
_CW = 1024
_NC_CH = _OUT_W // _CW
_G = 8
_RING = 8
_LEAD = 4


def _h_body(emb_ref, w1_ref, b1_ref, h_ref):
    h = jnp.dot(emb_ref[...], w1_ref[...], preferred_element_type=jnp.float32)
    h_ref[...] = jnp.tanh(h + b1_ref[...])


def _proj_body(h_ref, w2_ref, b2_ref, out_ref):
    out_ref[...] = (
        jnp.dot(h_ref[...], w2_ref[...], preferred_element_type=jnp.float32)
        + b2_ref[...]
    )


def _compute_table(emb_table, W1, b1, W2, b2):
    h = pl.pallas_call(
        _h_body,
        out_shape=jax.ShapeDtypeStruct((_VOCAB, _PHID), jnp.float32),
    )(emb_table, W1, b1.reshape(1, _PHID))

    return pl.pallas_call(
        _proj_body,
        grid=(_N_CB,),
        in_specs=[
            pl.BlockSpec((_VOCAB, _PHID), lambda j: (0, 0)),
            pl.BlockSpec((_PHID, _CB), lambda j: (0, j)),
            pl.BlockSpec((1, _CB), lambda j: (0, j)),
        ],
        out_specs=pl.BlockSpec((_VOCAB, _CB), lambda j: (0, j)),
        out_shape=jax.ShapeDtypeStruct((_VOCAB, _OUT_W), jnp.float32),
    )(h, W2, b2.reshape(1, _OUT_W))


def _gather(table, idx_flat, B, P):
    info = plsc.get_sparse_core_info()
    nc, ns = info.num_cores, info.num_subcores
    nw = nc * ns
    rpw = _N_ROWS // nw
    ngr = rpw // _G
    idx3 = idx_flat.reshape(nw, ngr, _G)

    mesh = plsc.VectorSubcoreMesh(core_axis_name="c", subcore_axis_name="s")

    @functools.partial(
        pl.kernel,
        mesh=mesh,
        out_type=jax.ShapeDtypeStruct((B, P, _OUT_W), jnp.float32),
        scratch_types=[
            pltpu.VMEM((ngr, _G), jnp.int32),
            pltpu.VMEM((_RING, _G, _CW), jnp.float32),
            pltpu.SemaphoreType.DMA((_RING,)),
            pltpu.SemaphoreType.DMA((_RING,)),
        ],
    )
    def gather_k(tab_hbm, idx_hbm, out_hbm, idx_v, bufs, sg, sw):
        wid = lax.axis_index("s") * nc + lax.axis_index("c")
        pltpu.sync_copy(idx_hbm.at[wid], idx_v)
        row0 = wid * rpw
        nit = ngr * _NC_CH

        def slot(it):
            return lax.rem(it, _RING)

        def gath(it):
            k = lax.div(it, _NC_CH)
            c = lax.rem(it, _NC_CH)
            coff = pl.multiple_of(_CW * c, _CW)
            return pltpu.make_async_copy(
                tab_hbm.at[idx_v.at[k], pl.ds(coff, _CW)],
                bufs.at[slot(it)], sg.at[slot(it)])

        def scat(it):
            k = lax.div(it, _NC_CH)
            c = lax.rem(it, _NC_CH)
            coff = pl.multiple_of(_CW * c, _CW)
            grow = row0 + _G * k
            b = lax.div(grow, P)
            p = pl.multiple_of(lax.rem(grow, P), _G)
            return pltpu.make_async_copy(
                bufs.at[slot(it)],
                out_hbm.at[b, pl.ds(p, _G), pl.ds(coff, _CW)],
                sw.at[slot(it)])

        def prologue(i, carry):
            gath(i).start()
            return carry

        lax.fori_loop(0, _LEAD, prologue, 0)

        def body(it, carry):
            @pl.when(it >= _LEAD)
            def _():
                scat(it - _LEAD).wait()

            @pl.when(it + _LEAD < nit)
            def _():
                gath(it + _LEAD).start()

            gath(it).wait()
            scat(it).start()
            return carry

        lax.fori_loop(0, nit, body, 0)

        def drain(t, carry):
            scat(nit - _LEAD + t).wait()
            return carry

        lax.fori_loop(0, _LEAD, drain, 0)

    return gather_k(table, idx3)


def kernel(emb_table, W1, b1, W2, b2, prefix):
    B, P = prefix.shape
    table = _compute_table(emb_table, W1, b1, W2, b2)
    return _gather(table, prefix.astype(jnp.int32).reshape(-1), B, P)


# speedup vs baseline: 1.0008x; 1.0008x over previous
import functools

import jax
import jax.numpy as jnp
from jax import lax
from jax.experimental import pallas as pl
from jax.experimental.pallas import tpu as pltpu
from jax.experimental.pallas import tpu_sc as plsc

_VOCAB = 200
_HID = 1024
_PHID = 512
_OUT_W = 49152
_N_ROWS = 6400
_CB = 2048
_N_CB = _OUT_W // _CB
_CW = 2048
_NC_CH = _OUT_W // _CW
_G = 8
_RING = 4
_LEAD = 2


def _h_body(emb_ref, w1_ref, b1_ref, h_ref):
    h = jnp.dot(emb_ref[...], w1_ref[...], preferred_element_type=jnp.float32)
    h_ref[...] = jnp.tanh(h + b1_ref[...])


def _proj_body(h_ref, w2_ref, b2_ref, out_ref):
    out_ref[...] = (
        jnp.dot(h_ref[...], w2_ref[...], preferred_element_type=jnp.float32)
        + b2_ref[...]
    )


def _compute_table(emb_table, W1, b1, W2, b2):
    h = pl.pallas_call(
        _h_body,
        out_shape=jax.ShapeDtypeStruct((_VOCAB, _PHID), jnp.float32),
    )(emb_table, W1, b1.reshape(1, _PHID))

    return pl.pallas_call(
        _proj_body,
        grid=(_N_CB,),
        in_specs=[
            pl.BlockSpec((_VOCAB, _PHID), lambda j: (0, 0)),
            pl.BlockSpec((_PHID, _CB), lambda j: (0, j)),
            pl.BlockSpec((1, _CB), lambda j: (0, j)),
        ],
        out_specs=pl.BlockSpec((_VOCAB, _CB), lambda j: (0, j)),
        out_shape=jax.ShapeDtypeStruct((_VOCAB, _OUT_W), jnp.float32),
    )(h, W2, b2.reshape(1, _OUT_W))


def _gather(table, idx_flat, B, P):
    info = plsc.get_sparse_core_info()
    nc, ns = info.num_cores, info.num_subcores
    nw = nc * ns
    rpw = _N_ROWS // nw
    ngr = rpw // _G
    idx3 = idx_flat.reshape(nw, ngr, _G)

    mesh = plsc.VectorSubcoreMesh(core_axis_name="c", subcore_axis_name="s")

    @functools.partial(
        pl.kernel,
        mesh=mesh,
        out_type=jax.ShapeDtypeStruct((B, P, _OUT_W), jnp.float32),
        scratch_types=[
            pltpu.VMEM((ngr, _G), jnp.int32),
            pltpu.VMEM((_RING, _G, _CW), jnp.float32),
            pltpu.SemaphoreType.DMA((_RING,)),
            pltpu.SemaphoreType.DMA((_RING,)),
        ],
    )
    def gather_k(tab_hbm, idx_hbm, out_hbm, idx_v, bufs, sg, sw):
        wid = lax.axis_index("s") * nc + lax.axis_index("c")
        pltpu.sync_copy(idx_hbm.at[wid], idx_v)
        row0 = wid * rpw
        nit = ngr * _NC_CH

        def slot(it):
            return lax.rem(it, _RING)

        def gath(it):
            k = lax.div(it, _NC_CH)
            c = lax.rem(it, _NC_CH)
            coff = pl.multiple_of(_CW * c, _CW)
            return pltpu.make_async_copy(
                tab_hbm.at[idx_v.at[k], pl.ds(coff, _CW)],
                bufs.at[slot(it)], sg.at[slot(it)])

        def scat(it):
            k = lax.div(it, _NC_CH)
            c = lax.rem(it, _NC_CH)
            coff = pl.multiple_of(_CW * c, _CW)
            grow = row0 + _G * k
            b = lax.div(grow, P)
            p = pl.multiple_of(lax.rem(grow, P), _G)
            return pltpu.make_async_copy(
                bufs.at[slot(it)],
                out_hbm.at[b, pl.ds(p, _G), pl.ds(coff, _CW)],
                sw.at[slot(it)])

        def prologue(i, carry):
            gath(i).start()
            return carry

        lax.fori_loop(0, _LEAD, prologue, 0)

        def body(it, carry):
            @pl.when(it >= _LEAD)
            def _():
                scat(it - _LEAD).wait()

            @pl.when(it + _LEAD < nit)
            def _():
                gath(it + _LEAD).start()

            gath(it).wait()
            scat(it).start()
            return carry

        lax.fori_loop(0, nit, body, 0)

        def drain(t, carry):
            scat(nit - _LEAD + t).wait()
            return carry

        lax.fori_loop(0, _LEAD, drain, 0)

    return gather_k(table, idx3)


def kernel(emb_table, W1, b1, W2, b2, prefix):
    B, P = prefix.shape
    table = _compute_table(emb_table, W1, b1, W2, b2)
    return _gather(table, prefix.astype(jnp.int32).reshape(-1), B, P)
